# baseline (speedup 1.0000x reference)
import functools

import jax
import jax.numpy as jnp
from jax.experimental import pallas as pl

B = 16
CIN = 263
WIDTH = 512
CODE_DIM = 512
NB_CODE = 1024
N_DEM_LAYERS = 8
DILS = (1, 3, 9)

_F32 = jnp.float32
_BF16 = jnp.bfloat16


def _mm(a, b):
    return jax.lax.dot_general(
        a.astype(_BF16), b.astype(_BF16), (((1,), (0,)), ((), ())),
        preferred_element_type=_F32)


def _mm_t(a, b):
    return jax.lax.dot_general(
        a.astype(_BF16), b.astype(_BF16), (((1,), (1,)), ((), ())),
        preferred_element_type=_F32)


def _mm_exact(a, b):
    return jax.lax.dot_general(
        a, b, (((1,), (0,)), ((), ())), preferred_element_type=_F32,
        precision=jax.lax.Precision.HIGHEST)


def _pad_t(h, lo, hi):
    zlo = jnp.zeros((h.shape[0], lo, h.shape[2]), _F32)
    zhi = jnp.zeros((h.shape[0], hi, h.shape[2]), _F32)
    return jnp.concatenate([zlo, h, zhi], axis=1)


def _conv3(h, w_ref, b_ref, dil, relu, idx=None):
    Tt = h.shape[1]
    Ci = h.shape[2]
    rp = _pad_t(h, dil, dil)
    p = jnp.concatenate(
        [rp[:, k * dil:k * dil + Tt, :] for k in range(3)], axis=2)
    w = w_ref[...] if idx is None else w_ref[idx]
    bb = b_ref[...] if idx is None else b_ref[idx]
    acc = _mm(p.reshape(-1, 3 * Ci), w) + bb
    if relu:
        acc = jnp.maximum(acc, 0.0)
    return acc.reshape(B, Tt, w_ref.shape[-1])


def _resnets_at(h, wr3_ref, br3_ref, wr1_ref, br1_ref, base):
    C = h.shape[2]
    Tt = h.shape[1]
    for j, d in enumerate(DILS):
        r = jnp.maximum(h, 0.0)
        rp = _pad_t(r, d, d)
        p = jnp.concatenate(
            [rp[:, k * d:k * d + Tt, :] for k in range(3)], axis=2)
        acc = _mm(p.reshape(-1, 3 * C), wr3_ref[base + j]) + br3_ref[base + j]
        r2 = jnp.maximum(acc, 0.0)
        r3 = _mm(r2, wr1_ref[base + j]) + br1_ref[base + j]
        h = h + r3.reshape(B, Tt, C)
    return h


def _enc_kernel(h_ref, wd_ref, bd_ref, wr3_ref, br3_ref, wr1_ref,
                br1_ref, o_ref):
    h = h_ref[...]
    for i in range(3):
        Tt = h.shape[1]
        C = h.shape[2]
        To = Tt // 2
        xp = _pad_t(h, 1, 3)
        xr = xp.reshape(B, (Tt + 4) // 2, 2, C)
        xe = xr[:, :, 0:1, :].reshape(B, (Tt + 4) // 2, C)
        xo = xr[:, :, 1:2, :].reshape(B, (Tt + 4) // 2, C)
        p = jnp.concatenate(
            [xe[:, 0:To, :], xo[:, 0:To, :], xe[:, 1:To + 1, :],
             xo[:, 1:To + 1, :]], axis=2)
        y = _mm(p.reshape(-1, 4 * C), wd_ref[i]) + bd_ref[i]
        h = y.reshape(B, To, C)
        h = _resnets_at(h, wr3_ref, br3_ref, wr1_ref, br1_ref, 3 * i)
    o_ref[...] = h


def _dec_kernel(h_ref, wr3_ref, br3_ref, wr1_ref, br1_ref, wu_ref,
                bu_ref, w1_ref, b1_ref, w2_ref, b2_ref, o_ref):
    h = h_ref[...]
    for i in range(3):
        h = _resnets_at(h, wr3_ref, br3_ref, wr1_ref, br1_ref, 3 * i)
        To = h.shape[1]
        C = h.shape[2]
        u = jnp.broadcast_to(h[:, :, None, :],
                             (B, To, 2, C)).reshape(B, 2 * To, C)
        h = _conv3(u, wu_ref, bu_ref, 1, False, idx=i)
    h = _conv3(h, w1_ref, b1_ref, 1, True)
    o_ref[...] = _conv3(h, w2_ref, b2_ref, 1, False)


def _mid_kernel(h_ref, we_ref, be_ref, cb_ref, dem_ref, wd_ref, bd_ref,
                o_ref, loss_ref, perp_ref):
    xe = _conv3(h_ref[...], we_ref, be_ref, 1, False)
    N = xe.shape[1]
    C = xe.shape[2]
    x = xe.reshape(B * N, C)
    M = B * N
    NB = cb_ref.shape[0]
    m = jnp.mean(x, axis=1, keepdims=True)
    xc = x - m
    v = jnp.mean(xc * xc, axis=1, keepdims=True)
    zf = xc / jnp.sqrt(v + 1e-5)
    cb = cb_ref[...]
    cb_n = jnp.sum(cb * cb, axis=1)[None, :]
    zf_n = jnp.sum(zf * zf, axis=1, keepdims=True)
    d2 = zf_n + cb_n - 2.0 * _mm_t(zf, cb)
    mn = jnp.min(d2, axis=1, keepdims=True)
    iota = jax.lax.broadcasted_iota(jnp.int32, (M, NB), 1)
    idx = jnp.min(jnp.where(d2 <= mn, iota, NB), axis=1, keepdims=True)
    onehot = (iota == idx).astype(_F32)
    zq = _mm_exact(onehot, cb)
    diff = zq - zf
    loss_ref[...] = 2.0 * jnp.mean(diff * diff, keepdims=True)
    em = jnp.mean(onehot, axis=0, keepdims=True)
    ent = jnp.sum(em * jnp.log(em + 1e-10), axis=1, keepdims=True)
    perp_ref[...] = jnp.exp(-ent)
    h = zf + (zq - zf)
    for i in range(N_DEM_LAYERS):
        h = h + jnp.maximum(_mm_t(h, dem_ref[i]), 0.0)
    o_ref[...] = _conv3(h.reshape(B, N, C), wd_ref, bd_ref, 1, True)


def _kmaj(w):
    return jnp.transpose(w, (2, 1, 0)).reshape(-1, w.shape[0]).astype(_BF16)


def _call(fn, args, out_shape):
    return pl.pallas_call(fn, out_shape=out_shape)(*args)


def _stage_weights(params):
    wr3 = jnp.stack([_kmaj(w) for (w, _, _, _) in params], axis=0)
    br3 = jnp.stack([b[None, :] for (_, b, _, _) in params], axis=0)
    wr1 = jnp.stack([jnp.transpose(w[:, :, 0]) for (_, _, w, _) in params],
                    axis=0).astype(_BF16)
    br1 = jnp.stack([b[None, :] for (_, _, _, b) in params], axis=0)
    return wr3, br3, wr1, br1


def kernel(x, enc_params, dec_params, dem_params, codebook):
    f32 = jax.ShapeDtypeStruct
    it = iter(enc_params)
    w, b = next(it)
    h0 = jax.lax.conv_general_dilated(
        jnp.transpose(x, (0, 2, 1)), w, window_strides=(1,),
        padding=[(1, 1)], dimension_numbers=('NCH', 'OIH', 'NCH'))
    h = jnp.transpose(jax.nn.relu(h0 + b[None, :, None]), (0, 2, 1))
    wds, bds, res = [], [], []
    for i in range(3):
        wd, bd = next(it)
        wds.append(_kmaj(wd))
        bds.append(bd[None, :])
        for j in range(3):
            w1, b1 = next(it)
            w2, b2 = next(it)
            res.append((w1, b1, w2, b2))
    wr3, br3, wr1, br1 = _stage_weights(res)
    h = _call(_enc_kernel,
              (h, jnp.stack(wds), jnp.stack(bds), wr3, br3, wr1, br1),
              f32((B, h.shape[1] // 8, WIDTH), _F32))
    we, be = next(it)

    dit = iter(dec_params)
    wdi, bdi = next(dit)
    dem_w = jnp.stack(dem_params).astype(_BF16)
    N = h.shape[1]
    h, loss, perp = _call(
        _mid_kernel,
        (h, _kmaj(we), be[None, :], codebook, dem_w,
         _kmaj(wdi), bdi[None, :]),
        (f32((B, N, CODE_DIM), _F32), f32((1, 1), _F32), f32((1, 1), _F32)))

    res, wus, bus = [], [], []
    for i in range(3):
        for j in range(3):
            w1, b1 = next(dit)
            w2, b2 = next(dit)
            res.append((w1, b1, w2, b2))
        wu, bu = next(dit)
        wus.append(_kmaj(wu))
        bus.append(bu[None, :])
    wr3, br3, wr1, br1 = _stage_weights(res)
    w1, b1 = next(dit)
    w2, b2 = next(dit)
    out_t = _call(_dec_kernel,
                  (h, wr3, br3, wr1, br1, jnp.stack(wus), jnp.stack(bus),
                   _kmaj(w1), b1[None, :], _kmaj(w2), b2[None, :]),
                  f32((B, h.shape[1] * 8, CIN), _F32))
    out = jnp.transpose(out_t, (0, 2, 1))
    return out, loss[0, 0], perp[0, 0]
